# Initial kernel scaffold; baseline (speedup 1.0000x reference)
import jax
import jax.numpy as jnp
from jax.experimental import pallas as pl


def kernel(vmem, vlastmem, labels):
    raise NotImplementedError("write your pallas kernel here")


# speedup vs baseline: 30.2509x; 30.2509x over previous
import functools

import jax
import jax.numpy as jnp
from jax.experimental import pallas as pl

_C = 5
_T = 512
_ROWS_PER_BLOCK = 256


def _scan(x, op, fill):
    n = x.shape[-1]
    s = 1
    while s < n:
        pad = jnp.full(x.shape[:-1] + (s,), fill, x.dtype)
        shifted = jnp.concatenate([pad, x[..., :-s]], axis=-1)
        x = op(x, shifted)
        s *= 2
    return x


def _shift_r(x, s, fill):
    pad = jnp.full(x.shape[:-1] + (s,), fill, x.dtype)
    return jnp.concatenate([pad, x[..., :-s]], axis=-1)


def _stca_block(v_ref, tgt_ref, spike_ref, loss_ref):
    v = v_ref[...]
    R, T = v.shape
    t_idx = jax.lax.broadcasted_iota(jnp.int32, (R, T), 1)

    is_pos = v >= 0.0
    ip = is_pos.astype(jnp.int32)
    P = _scan(ip, jnp.add, jnp.int32(0))

    prev_cnt = _shift_r(P, 1, jnp.int32(0)) - _shift_r(P, _C + 1, jnp.int32(0))
    start = is_pos & (prev_cnt == 0)
    p_last = jnp.broadcast_to(P[:, T - 1:], (R, _C))
    next_cnt = jnp.concatenate([P[:, _C:], p_last], axis=-1) - P
    end = is_pos & (next_cnt == 0)

    s_ff = _scan(jnp.where(start, P - 1, jnp.int32(-1)), jnp.maximum,
                 jnp.int32(-1))
    cnt_at_end = P - s_ff

    big = jnp.int32(2 ** 30)
    score = jnp.where(end, cnt_at_end * T + t_idx, big)
    m = jnp.min(score, axis=-1)
    t_star = jnp.bitwise_and(m, T - 1)

    first = jnp.max(jnp.where(start & (t_idx <= t_star[:, None]), t_idx, -1),
                    axis=-1)
    span_pos = (t_idx >= first[:, None]) & (t_idx <= t_star[:, None]) & (v > 0.0)
    psum = jnp.sum(jnp.where(span_pos, v, 0.0), axis=-1)
    pcnt = jnp.sum(span_pos.astype(jnp.float32), axis=-1)
    contrib = jnp.where(pcnt > 0.0, psum / jnp.maximum(pcnt, 1.0), 0.0)

    vmax = jnp.max(v, axis=-1)
    ncl = jnp.sum(start.astype(jnp.float32), axis=-1)
    spiked = ncl > 0.0
    tgt = tgt_ref[0, 0, :] != 0
    rowloss = jnp.where(tgt & ~spiked, -vmax,
                        jnp.where((~tgt) & spiked, contrib, 0.0))

    spike_ref[0, 0, :] = ncl

    @pl.when(pl.program_id(0) == 0)
    def _init():
        loss_ref[...] = jnp.zeros((1, 1), jnp.float32)

    loss_ref[...] += jnp.sum(rowloss).reshape(1, 1)


@functools.partial(jax.jit, static_argnames=())
def _run(vmem, labels):
    B, N, T = vmem.shape
    rows = B * N
    nblk = rows // _ROWS_PER_BLOCK
    v2 = vmem.reshape(rows, T)
    tgt = (labels[:, None] == jnp.arange(N, dtype=labels.dtype)[None, :])
    tgt = tgt.reshape(nblk, 1, _ROWS_PER_BLOCK).astype(jnp.int32)

    spike, loss = pl.pallas_call(
        _stca_block,
        grid=(nblk,),
        in_specs=[
            pl.BlockSpec((_ROWS_PER_BLOCK, T), lambda i: (i, 0)),
            pl.BlockSpec((1, 1, _ROWS_PER_BLOCK), lambda i: (i, 0, 0)),
        ],
        out_specs=[
            pl.BlockSpec((1, 1, _ROWS_PER_BLOCK), lambda i: (i, 0, 0)),
            pl.BlockSpec((1, 1), lambda i: (0, 0)),
        ],
        out_shape=[
            jax.ShapeDtypeStruct((nblk, 1, _ROWS_PER_BLOCK), jnp.float32),
            jax.ShapeDtypeStruct((1, 1), jnp.float32),
        ],
    )(v2, tgt)
    return loss[0, 0], spike.reshape(B, N)


def kernel(vmem, vlastmem, labels):
    del vlastmem
    return _run(vmem, labels)


# speedup vs baseline: 39.4710x; 1.3048x over previous
import functools

import jax
import jax.numpy as jnp
from jax.experimental import pallas as pl
from jax.experimental.pallas import tpu as pltpu

_C = 5
_T = 512
_ROWS_PER_BLOCK = 256


def _cummax(x, fill):
    n = x.shape[-1]
    s = 1
    while s < n:
        pad = jnp.full(x.shape[:-1] + (s,), fill, x.dtype)
        shifted = jnp.concatenate([pad, x[..., :-s]], axis=-1)
        x = jnp.maximum(x, shifted)
        s *= 2
    return x


def _stca_block(v_ref, tgt_ref, spike_ref, loss_ref, m_ref):
    T = _T

    @pl.when(pl.program_id(0) == 0)
    def _init_mats():
        a = jax.lax.broadcasted_iota(jnp.int32, (T, T), 0)
        b = jax.lax.broadcasted_iota(jnp.int32, (T, T), 1)
        m_ref[:, :T] = (a <= b).astype(jnp.bfloat16)
        m_ref[:, T:2 * T] = ((a >= b - _C) & (a <= b - 1)).astype(jnp.bfloat16)
        m_ref[:, 2 * T:] = ((a >= b + 1) & (a <= b + _C)).astype(jnp.bfloat16)
        loss_ref[...] = jnp.zeros((1, 1), jnp.float32)

    v = v_ref[...]
    R = v.shape[0]
    t_idx = jax.lax.broadcasted_iota(jnp.int32, (R, T), 1)

    is_pos = v >= 0.0
    sums = jax.lax.dot_general(
        is_pos.astype(jnp.bfloat16), m_ref[...],
        (((1,), (0,)), ((), ())), preferred_element_type=jnp.float32)
    P = sums[:, :T]
    prev_cnt = sums[:, T:2 * T]
    next_cnt = sums[:, 2 * T:]

    start = is_pos & (prev_cnt == 0.0)
    end = is_pos & (next_cnt == 0.0)

    s_ff = _cummax(jnp.where(start, P - 1.0, -1.0), jnp.float32(-1.0))
    cnt_at_end = P - s_ff

    big = jnp.float32(2 ** 30)
    t_f = t_idx.astype(jnp.float32)
    score = jnp.where(end, cnt_at_end * T + t_f, big)
    m = jnp.min(score, axis=-1)
    t_star = jnp.bitwise_and(m.astype(jnp.int32), T - 1)

    first = jnp.max(jnp.where(start & (t_idx <= t_star[:, None]), t_idx, -1),
                    axis=-1)
    span_pos = (t_idx >= first[:, None]) & (t_idx <= t_star[:, None]) & (v > 0.0)
    psum = jnp.sum(jnp.where(span_pos, v, 0.0), axis=-1)
    pcnt = jnp.sum(span_pos.astype(jnp.float32), axis=-1)
    contrib = jnp.where(pcnt > 0.0, psum / jnp.maximum(pcnt, 1.0), 0.0)

    vmax = jnp.max(v, axis=-1)
    ncl = jnp.sum(start.astype(jnp.float32), axis=-1)
    spiked = ncl > 0.0
    tgt = tgt_ref[0, 0, :] != 0
    rowloss = jnp.where(tgt & ~spiked, -vmax,
                        jnp.where((~tgt) & spiked, contrib, 0.0))

    spike_ref[0, 0, :] = ncl
    loss_ref[...] += jnp.sum(rowloss).reshape(1, 1)


@functools.partial(jax.jit, static_argnames=())
def _run(vmem, labels):
    B, N, T = vmem.shape
    rows = B * N
    nblk = rows // _ROWS_PER_BLOCK
    v2 = vmem.reshape(rows, T)
    tgt = (labels[:, None] == jnp.arange(N, dtype=labels.dtype)[None, :])
    tgt = tgt.reshape(nblk, 1, _ROWS_PER_BLOCK).astype(jnp.int32)

    spike, loss = pl.pallas_call(
        _stca_block,
        grid=(nblk,),
        in_specs=[
            pl.BlockSpec((_ROWS_PER_BLOCK, T), lambda i: (i, 0)),
            pl.BlockSpec((1, 1, _ROWS_PER_BLOCK), lambda i: (i, 0, 0)),
        ],
        out_specs=[
            pl.BlockSpec((1, 1, _ROWS_PER_BLOCK), lambda i: (i, 0, 0)),
            pl.BlockSpec((1, 1), lambda i: (0, 0)),
        ],
        out_shape=[
            jax.ShapeDtypeStruct((nblk, 1, _ROWS_PER_BLOCK), jnp.float32),
            jax.ShapeDtypeStruct((1, 1), jnp.float32),
        ],
        scratch_shapes=[pltpu.VMEM((T, 3 * T), jnp.bfloat16)],
    )(v2, tgt)
    return loss[0, 0], spike.reshape(B, N)


def kernel(vmem, vlastmem, labels):
    del vlastmem
    return _run(vmem, labels)


# speedup vs baseline: 41.6191x; 1.0544x over previous
import functools

import jax
import jax.numpy as jnp
from jax.experimental import pallas as pl
from jax.experimental.pallas import tpu as pltpu

_C = 5
_T = 512
_ROWS_PER_BLOCK = 512


def _cummax(x, fill):
    n = x.shape[-1]
    s = 1
    while s < n:
        pad = jnp.full(x.shape[:-1] + (s,), fill, x.dtype)
        shifted = jnp.concatenate([pad, x[..., :-s]], axis=-1)
        x = jnp.maximum(x, shifted)
        s *= 2
    return x


def _stca_block(v_ref, tgt_ref, spike_ref, loss_ref, m_ref):
    T = _T

    @pl.when(pl.program_id(0) == 0)
    def _init_mats():
        a = jax.lax.broadcasted_iota(jnp.int32, (T, T), 0)
        b = jax.lax.broadcasted_iota(jnp.int32, (T, T), 1)
        m_ref[:, :T] = (a <= b).astype(jnp.bfloat16)
        m_ref[:, T:2 * T] = ((a >= b - _C) & (a <= b - 1)).astype(jnp.bfloat16)
        m_ref[:, 2 * T:] = ((a >= b + 1) & (a <= b + _C)).astype(jnp.bfloat16)
        loss_ref[...] = jnp.zeros((1, 1), jnp.float32)

    v = v_ref[...]
    R = v.shape[0]
    t_idx = jax.lax.broadcasted_iota(jnp.int32, (R, T), 1)

    is_pos = v >= 0.0
    sums = jax.lax.dot_general(
        is_pos.astype(jnp.bfloat16), m_ref[...],
        (((1,), (0,)), ((), ())), preferred_element_type=jnp.float32)
    P = sums[:, :T]
    prev_cnt = sums[:, T:2 * T]
    next_cnt = sums[:, 2 * T:]

    start = is_pos & (prev_cnt == 0.0)
    end = is_pos & (next_cnt == 0.0)

    s_ff = _cummax(jnp.where(start, P - 1.0, -1.0), jnp.float32(-1.0))
    cnt_at_end = P - s_ff

    big = jnp.float32(2 ** 30)
    t_f = t_idx.astype(jnp.float32)
    score = jnp.where(end, cnt_at_end * T + t_f, big)
    m = jnp.min(score, axis=-1)
    t_star = jnp.bitwise_and(m.astype(jnp.int32), T - 1)

    first = jnp.max(jnp.where(start & (t_idx <= t_star[:, None]), t_idx, -1),
                    axis=-1)
    span_pos = (t_idx >= first[:, None]) & (t_idx <= t_star[:, None]) & (v > 0.0)
    psum = jnp.sum(jnp.where(span_pos, v, 0.0), axis=-1)
    pcnt = jnp.sum(span_pos.astype(jnp.float32), axis=-1)
    contrib = jnp.where(pcnt > 0.0, psum / jnp.maximum(pcnt, 1.0), 0.0)

    vmax = jnp.max(v, axis=-1)
    ncl = jnp.sum(start.astype(jnp.float32), axis=-1)
    spiked = ncl > 0.0
    tgt = tgt_ref[0, 0, :] != 0
    rowloss = jnp.where(tgt & ~spiked, -vmax,
                        jnp.where((~tgt) & spiked, contrib, 0.0))

    spike_ref[0, 0, :] = ncl
    loss_ref[...] += jnp.sum(rowloss).reshape(1, 1)


@functools.partial(jax.jit, static_argnames=())
def _run(vmem, labels):
    B, N, T = vmem.shape
    rows = B * N
    nblk = rows // _ROWS_PER_BLOCK
    v2 = vmem.reshape(rows, T)
    tgt = (labels[:, None] == jnp.arange(N, dtype=labels.dtype)[None, :])
    tgt = tgt.reshape(nblk, 1, _ROWS_PER_BLOCK).astype(jnp.int32)

    spike, loss = pl.pallas_call(
        _stca_block,
        grid=(nblk,),
        in_specs=[
            pl.BlockSpec((_ROWS_PER_BLOCK, T), lambda i: (i, 0)),
            pl.BlockSpec((1, 1, _ROWS_PER_BLOCK), lambda i: (i, 0, 0)),
        ],
        out_specs=[
            pl.BlockSpec((1, 1, _ROWS_PER_BLOCK), lambda i: (i, 0, 0)),
            pl.BlockSpec((1, 1), lambda i: (0, 0)),
        ],
        out_shape=[
            jax.ShapeDtypeStruct((nblk, 1, _ROWS_PER_BLOCK), jnp.float32),
            jax.ShapeDtypeStruct((1, 1), jnp.float32),
        ],
        scratch_shapes=[pltpu.VMEM((T, 3 * T), jnp.bfloat16)],
    )(v2, tgt)
    return loss[0, 0], spike.reshape(B, N)


def kernel(vmem, vlastmem, labels):
    del vlastmem
    return _run(vmem, labels)


# speedup vs baseline: 42.5001x; 1.0212x over previous
import functools

import jax
import jax.numpy as jnp
from jax.experimental import pallas as pl
from jax.experimental.pallas import tpu as pltpu

_C = 5
_T = 512
_ROWS_PER_BLOCK = 1024


def _cummax(x, fill):
    n = x.shape[-1]
    s = 1
    while s < n:
        pad = jnp.full(x.shape[:-1] + (s,), fill, x.dtype)
        shifted = jnp.concatenate([pad, x[..., :-s]], axis=-1)
        x = jnp.maximum(x, shifted)
        s *= 2
    return x


def _stca_block(v_ref, tgt_ref, spike_ref, loss_ref, m_ref):
    T = _T

    @pl.when(pl.program_id(0) == 0)
    def _init_mats():
        a = jax.lax.broadcasted_iota(jnp.int32, (T, T), 0)
        b = jax.lax.broadcasted_iota(jnp.int32, (T, T), 1)
        m_ref[:, :T] = (a <= b).astype(jnp.bfloat16)
        m_ref[:, T:2 * T] = ((a >= b - _C) & (a <= b - 1)).astype(jnp.bfloat16)
        m_ref[:, 2 * T:] = ((a >= b + 1) & (a <= b + _C)).astype(jnp.bfloat16)
        loss_ref[...] = jnp.zeros((1, 1), jnp.float32)

    v = v_ref[...]
    R = v.shape[0]
    t_idx = jax.lax.broadcasted_iota(jnp.int32, (R, T), 1)

    is_pos = v >= 0.0
    sums = jax.lax.dot_general(
        is_pos.astype(jnp.bfloat16), m_ref[...],
        (((1,), (0,)), ((), ())), preferred_element_type=jnp.float32)
    P = sums[:, :T]
    prev_cnt = sums[:, T:2 * T]
    next_cnt = sums[:, 2 * T:]

    start = is_pos & (prev_cnt == 0.0)
    end = is_pos & (next_cnt == 0.0)

    s_ff = _cummax(jnp.where(start, P - 1.0, -1.0), jnp.float32(-1.0))
    cnt_at_end = P - s_ff

    big = jnp.float32(2 ** 30)
    t_f = t_idx.astype(jnp.float32)
    score = jnp.where(end, cnt_at_end * T + t_f, big)
    m = jnp.min(score, axis=-1)
    t_star = jnp.bitwise_and(m.astype(jnp.int32), T - 1)

    first = jnp.max(jnp.where(start & (t_idx <= t_star[:, None]), t_idx, -1),
                    axis=-1)
    span_pos = (t_idx >= first[:, None]) & (t_idx <= t_star[:, None]) & (v > 0.0)
    psum = jnp.sum(jnp.where(span_pos, v, 0.0), axis=-1)
    pcnt = jnp.sum(span_pos.astype(jnp.float32), axis=-1)
    contrib = jnp.where(pcnt > 0.0, psum / jnp.maximum(pcnt, 1.0), 0.0)

    vmax = jnp.max(v, axis=-1)
    ncl = jnp.sum(start.astype(jnp.float32), axis=-1)
    spiked = ncl > 0.0
    tgt = tgt_ref[0, 0, :] != 0
    rowloss = jnp.where(tgt & ~spiked, -vmax,
                        jnp.where((~tgt) & spiked, contrib, 0.0))

    spike_ref[0, 0, :] = ncl
    loss_ref[...] += jnp.sum(rowloss).reshape(1, 1)


@functools.partial(jax.jit, static_argnames=())
def _run(vmem, labels):
    B, N, T = vmem.shape
    rows = B * N
    nblk = rows // _ROWS_PER_BLOCK
    v2 = vmem.reshape(rows, T)
    tgt = (labels[:, None] == jnp.arange(N, dtype=labels.dtype)[None, :])
    tgt = tgt.reshape(nblk, 1, _ROWS_PER_BLOCK).astype(jnp.int32)

    spike, loss = pl.pallas_call(
        _stca_block,
        grid=(nblk,),
        in_specs=[
            pl.BlockSpec((_ROWS_PER_BLOCK, T), lambda i: (i, 0)),
            pl.BlockSpec((1, 1, _ROWS_PER_BLOCK), lambda i: (i, 0, 0)),
        ],
        out_specs=[
            pl.BlockSpec((1, 1, _ROWS_PER_BLOCK), lambda i: (i, 0, 0)),
            pl.BlockSpec((1, 1), lambda i: (0, 0)),
        ],
        out_shape=[
            jax.ShapeDtypeStruct((nblk, 1, _ROWS_PER_BLOCK), jnp.float32),
            jax.ShapeDtypeStruct((1, 1), jnp.float32),
        ],
        scratch_shapes=[pltpu.VMEM((T, 3 * T), jnp.bfloat16)],
    )(v2, tgt)
    return loss[0, 0], spike.reshape(B, N)


def kernel(vmem, vlastmem, labels):
    del vlastmem
    return _run(vmem, labels)
